# baseline (speedup 1.0000x reference)
import functools

import numpy as np
import jax
import jax.numpy as jnp
from jax import lax
from jax.experimental import pallas as pl
from jax.experimental.pallas import tpu as pltpu
from jax.experimental.pallas import tpu_sc as plsc

_NF = 26
_NFP = 32
_ED = 16
_FD = 260000
_B = 4096
_NP = (_NF * (_NF - 1)) // 2
_NPP = 336
_OFF = np.arange(_NF, dtype=np.int32) * 10000

_pairs = np.array([(f, g) for f in range(_NF - 1) for g in range(f + 1, _NF)],
                  dtype=np.int32)
_PF = _pairs[:, 0]
_PG = _pairs[:, 1]

_NW = 32
_RPS = _B // _NW
_CH = 128
_NCHF = _NPP // _CH
_CHT = _NPP - _NCHF * _CH


def _static_tables():
    colsa = np.zeros(_NPP, np.int32)
    addsa = np.zeros(_NPP, np.int32)
    colsb = np.zeros(_NPP, np.int32)
    addsb = np.zeros(_NPP, np.int32)
    colsa[:_NP] = _PG
    addsa[:_NP] = _OFF[_PG] + _PF * _FD
    colsb[:_NP] = _PF
    addsb[:_NP] = _OFF[_PF] + _PG * _FD
    offc = np.zeros(_NFP, np.int32)
    offc[:_NF] = _OFF
    return (jnp.asarray(colsa), jnp.asarray(addsa),
            jnp.asarray(colsb), jnp.asarray(addsb), jnp.asarray(offc))


def _sc_body(ffm_h, xp_h, colsa_h, addsa_h, colsb_h, addsb_h, offs_h,
             emb_h, lw_h,
             inter_o, hrows_o, lwv_o,
             colsa_v, addsa_v, colsb_v, addsb_v, offs_v,
             xf_v, iaba_v, ioffa_v, iabb_v, ioffb_v,
             ieq_v, eo_v, lrow_v, lcol_v,
             bufa_v, bufb_v, out_v, ebuf_v, lbuf_v, hc_v, lwc_v,
             sem, sem2):
    nc = 2
    wid = lax.axis_index("s") * nc + lax.axis_index("c")
    pltpu.sync_copy(colsa_h, colsa_v)
    pltpu.sync_copy(addsa_h, addsa_v)
    pltpu.sync_copy(colsb_h, colsb_v)
    pltpu.sync_copy(addsb_h, addsb_v)
    pltpu.sync_copy(offs_h, offs_v)
    iot = lax.iota(jnp.int32, 16)

    def step(t, carry):
        row = wid * _RPS + t
        x_off = pl.multiple_of(row * _NFP, _NFP)
        pltpu.sync_copy(xp_h.at[pl.ds(x_off, _NFP)], xf_v)

        def bld_ie(i, c2):
            s = pl.ds(pl.multiple_of(i * 16, 16), 16)
            xo = xf_v[s] + offs_v[s]
            ieq_v[s] = xo >> 3
            eo_v[s] = (xo & 7) << 4
            lrow_v[s] = xo >> 7
            lcol_v[s] = xo & 127
            return c2

        lax.fori_loop(0, _NFP // 16, bld_ie, 0)

        def bld_iab(i, c2):
            s = pl.ds(pl.multiple_of(i * 16, 16), 16)
            rawa = plsc.load_gather(xf_v, [colsa_v[s]]) + addsa_v[s]
            iaba_v[s] = rawa >> 3
            ioffa_v[s] = (rawa & 7) << 4
            rawb = plsc.load_gather(xf_v, [colsb_v[s]]) + addsb_v[s]
            iabb_v[s] = rawb >> 3
            ioffb_v[s] = (rawb & 7) << 4
            return c2

        lax.fori_loop(0, _NPP // 16, bld_iab, 0)

        cps = []
        for c in range(_NCHF):
            cps.append(pltpu.async_copy(
                ffm_h.at[iaba_v.at[pl.ds(c * _CH, _CH)]],
                bufa_v.at[pl.ds(c * _CH, _CH)], sem))
            cps.append(pltpu.async_copy(
                ffm_h.at[iabb_v.at[pl.ds(c * _CH, _CH)]],
                bufb_v.at[pl.ds(c * _CH, _CH)], sem))
        cps.append(pltpu.async_copy(
            ffm_h.at[iaba_v.at[pl.ds(_NCHF * _CH, _CHT)]],
            bufa_v.at[pl.ds(_NCHF * _CH, _CHT)], sem))
        cps.append(pltpu.async_copy(
            ffm_h.at[iabb_v.at[pl.ds(_NCHF * _CH, _CHT)]],
            bufb_v.at[pl.ds(_NCHF * _CH, _CHT)], sem))
        cps.append(pltpu.async_copy(emb_h.at[ieq_v], ebuf_v, sem2))
        cps.append(pltpu.async_copy(lw_h.at[lrow_v], lbuf_v, sem2))
        for cp in cps:
            cp.wait()

        def prod(q, c2):
            sq = pl.ds(pl.multiple_of(q * 16, 16), 16)
            rowv = iot + q * 16
            offa = ioffa_v[sq]
            offb = ioffb_v[sq]
            for e in range(_ED):
                av = plsc.load_gather(bufa_v, [rowv, offa + e])
                bv = plsc.load_gather(bufb_v, [rowv, offb + e])
                o = pl.ds(pl.multiple_of(q * 16 + e * _NPP, 16), 16)
                out_v[o] = av * bv
            return c2

        lax.fori_loop(0, _NPP // 16, prod, 0)

        for c in range(_NFP // 16):
            sq = pl.ds(c * 16, 16)
            rowv = iot + c * 16
            eo = eo_v[sq]
            for e in range(_ED):
                hc_v[pl.ds(e * _NFP + c * 16, 16)] = (
                    plsc.load_gather(ebuf_v, [rowv, eo + e]))

        for c in range(_NFP // 16):
            s = pl.ds(c * 16, 16)
            lwc_v[s] = plsc.load_gather(lbuf_v, [iot + (16 * c), lcol_v[s]])

        int_off = pl.multiple_of(row * _ED * _NPP, _ED * _NPP)
        pltpu.sync_copy(out_v, inter_o.at[pl.ds(int_off, _ED * _NPP)])
        h_off = pl.multiple_of(row * _NFP * _ED, _NFP * _ED)
        pltpu.sync_copy(hc_v, hrows_o.at[pl.ds(h_off, _NFP * _ED)])
        pltpu.sync_copy(lwc_v, lwv_o.at[pl.ds(x_off, _NFP)])
        return carry

    lax.fori_loop(0, _RPS, step, 0)


_sc_call_cache = []


def _sc_call(*args):
    if not _sc_call_cache:
        _sc_call_cache.append(pl.kernel(
            _sc_body,
            out_type=[
                jax.ShapeDtypeStruct((_B * _ED * _NPP,), jnp.float32),
                jax.ShapeDtypeStruct((_B * _NFP * _ED,), jnp.float32),
                jax.ShapeDtypeStruct((_B * _NFP,), jnp.float32),
            ],
            mesh=plsc.VectorSubcoreMesh(core_axis_name="c",
                                        subcore_axis_name="s"),
            compiler_params=pltpu.CompilerParams(use_tc_tiling_on_sc=True,
                                                 needs_layout_passes=False),
            scratch_types=[
                pltpu.VMEM((_NPP,), jnp.int32),
                pltpu.VMEM((_NPP,), jnp.int32),
                pltpu.VMEM((_NPP,), jnp.int32),
                pltpu.VMEM((_NPP,), jnp.int32),
                pltpu.VMEM((_NFP,), jnp.int32),
                pltpu.VMEM((_NFP,), jnp.int32),
                pltpu.VMEM((_NPP,), jnp.int32),
                pltpu.VMEM((_NPP,), jnp.int32),
                pltpu.VMEM((_NPP,), jnp.int32),
                pltpu.VMEM((_NPP,), jnp.int32),
                pltpu.VMEM((_NFP,), jnp.int32),
                pltpu.VMEM((_NFP,), jnp.int32),
                pltpu.VMEM((_NFP,), jnp.int32),
                pltpu.VMEM((_NFP,), jnp.int32),
                pltpu.VMEM((_NPP, 128), jnp.float32),
                pltpu.VMEM((_NPP, 128), jnp.float32),
                pltpu.VMEM((_ED * _NPP,), jnp.float32),
                pltpu.VMEM((_NFP, 128), jnp.float32),
                pltpu.VMEM((_NFP, 128), jnp.float32),
                pltpu.VMEM((_NFP * _ED,), jnp.float32),
                pltpu.VMEM((_NFP,), jnp.float32),
                pltpu.SemaphoreType.DMA,
                pltpu.SemaphoreType.DMA,
            ],
        ))
    return _sc_call_cache[0](*args)


_BB = 512


def _tc_body(inter_r, h_r, lwv_r, lb_r,
             w1_r, b1_r, w2_r, b2_r,
             wf1f_r, wf1i_r, wf1d_r, bf1_r,
             wf2_r, bf2_r, wo_r, bo_r, out_r):
    dot = functools.partial(jnp.dot, preferred_element_type=jnp.float32,
                            precision=lax.Precision.HIGHEST)
    fmask = (lax.broadcasted_iota(jnp.int32, (_BB, _NFP), 1) < _NF)
    first = (jnp.sum(jnp.where(fmask, lwv_r[...], 0.0), axis=1, keepdims=True)
             + lb_r[0, 0])
    d1 = jnp.maximum(dot(h_r[...], w1_r[...]) + b1_r[...], 0.0)
    d2 = jnp.maximum(dot(d1, w2_r[...]) + b2_r[...], 0.0)
    t = (dot(inter_r[...], wf1i_r[...]) + dot(d2, wf1d_r[...])
         + first * wf1f_r[...] + bf1_r[...])
    h2 = jnp.maximum(t, 0.0)
    h3 = jnp.maximum(dot(h2, wf2_r[...]) + bf2_r[...], 0.0)
    out_r[...] = dot(h3, wo_r[...]) + bo_r[...]


def _full(shape):
    return pl.BlockSpec(shape, lambda i: (0, 0))


_tc_call = pl.pallas_call(
    _tc_body,
    grid=(_B // _BB,),
    in_specs=[
        pl.BlockSpec((_BB, _ED * _NPP), lambda i: (i, 0)),
        pl.BlockSpec((_BB, _NFP * _ED), lambda i: (i, 0)),
        pl.BlockSpec((_BB, _NFP), lambda i: (i, 0)),
        _full((1, 1)),
        _full((_NFP * _ED, 64)), _full((1, 64)),
        _full((64, 64)), _full((1, 64)),
        _full((1, 64)), _full((_ED * _NPP, 64)), _full((64, 64)), _full((1, 64)),
        _full((64, 32)), _full((1, 32)),
        _full((32, 1)), _full((1, 1)),
    ],
    out_specs=pl.BlockSpec((_BB, 1), lambda i: (i, 0)),
    out_shape=jax.ShapeDtypeStruct((_B, 1), jnp.float32),
)


def _fold_bn(W, b, g, be, m, v):
    s = g * lax.rsqrt(v + 1e-5)
    return W * s[None, :], ((b - m) * s + be)[None, :]


def kernel(x, linear_w, linear_b, emb, ffm,
           W1, b1, g1, be1, m1, v1, W2, b2, g2, be2, m2, v2,
           Wf1, bf1, gf1, bef1, mf1, vf1, Wf2, bf2, gf2, bef2, mf2, vf2,
           Wout, bout):
    xp = jnp.pad(x, ((0, 0), (0, _NFP - _NF))).reshape(-1)
    colsa, addsa, colsb, addsb, offs = _static_tables()
    ffm128 = ffm.reshape(_NF * _FD * _ED // 128, 128)
    emb128 = emb.reshape(_FD * _ED // 128, 128)
    lw128 = jnp.pad(linear_w.reshape(-1), (0, 96)).reshape(-1, 128)

    inter_f, hrows, lwv = _sc_call(ffm128, xp, colsa, addsa, colsb, addsb,
                                   offs, emb128, lw128)

    W1p, b1p = _fold_bn(W1, b1, g1, be1, m1, v1)
    W1a = jnp.pad(W1p.reshape(_NF, _ED, 64),
                  ((0, _NFP - _NF), (0, 0), (0, 0)))
    W1a = W1a.transpose(1, 0, 2).reshape(_NFP * _ED, 64)
    W2p, b2p = _fold_bn(W2, b2, g2, be2, m2, v2)
    Wf1p, bf1p = _fold_bn(Wf1, bf1, gf1, bef1, mf1, vf1)
    Wf1i = jnp.pad(Wf1p[1:1 + _NP * _ED, :].reshape(_NP, _ED, 64),
                   ((0, _NPP - _NP), (0, 0), (0, 0)))
    Wf1i = Wf1i.transpose(1, 0, 2).reshape(_ED * _NPP, 64)
    Wf2p, bf2p = _fold_bn(Wf2, bf2, gf2, bef2, mf2, vf2)

    out2d = _tc_call(
        inter_f.reshape(_B, _ED * _NPP),
        hrows.reshape(_B, _NFP * _ED),
        lwv.reshape(_B, _NFP),
        linear_b.reshape(1, 1),
        W1a, b1p, W2p, b2p,
        Wf1p[0:1, :], Wf1i, Wf1p[1 + _NP * _ED:, :], bf1p,
        Wf2p, bf2p, Wout, bout.reshape(1, 1),
    )
    return out2d[:, 0]


# speedup vs baseline: 1.3942x; 1.3942x over previous
import functools

import numpy as np
import jax
import jax.numpy as jnp
from jax import lax
from jax.experimental import pallas as pl
from jax.experimental.pallas import tpu as pltpu
from jax.experimental.pallas import tpu_sc as plsc

_NF = 26
_NFP = 32
_ED = 16
_FD = 260000
_B = 4096
_NP = (_NF * (_NF - 1)) // 2
_NPP = 336
_OFF = np.arange(_NF, dtype=np.int32) * 10000

_pairs = np.array([(f, g) for f in range(_NF - 1) for g in range(f + 1, _NF)],
                  dtype=np.int32)
_PF = _pairs[:, 0]
_PG = _pairs[:, 1]

_NW = 32
_RPS = _B // _NW
_RT = 4
_STEPS = _RPS // _RT
_IAB = _RT * 2 * _NPP
_CH = 128
_NCH = _IAB // _CH
_IE = _RT * _NFP


def _static_tables():
    col1 = np.zeros(2 * _NPP, np.int32)
    add1 = np.zeros(2 * _NPP, np.int32)
    col1[0:650:2] = _PG
    add1[0:650:2] = _OFF[_PG] + _PF * _FD
    col1[1:650:2] = _PF
    add1[1:650:2] = _OFF[_PF] + _PG * _FD
    cols = np.concatenate([col1 + 32 * r for r in range(_RT)])
    adds = np.concatenate([add1 for _ in range(_RT)])
    offc = np.zeros(_NFP, np.int32)
    offc[:_NF] = _OFF
    offs = np.tile(offc, _RT)
    return jnp.asarray(cols), jnp.asarray(adds), jnp.asarray(offs)


def _sc_body(ffm_f, xp_h, cols_h, adds_h, offs_h, emb_h, lw16_h,
             inter_o, hrows_o, lwg_o,
             cols_v, adds_v, offs_v,
             xf_v, iab_v, ie_v, iq_v, buf_v, out_v, ebuf_v, lbuf_v,
             sem, sem2):
    nc = 2
    wid = lax.axis_index("s") * nc + lax.axis_index("c")
    pltpu.sync_copy(cols_h, cols_v)
    pltpu.sync_copy(adds_h, adds_v)
    pltpu.sync_copy(offs_h, offs_v)

    def step(t, carry):
        row0 = wid * _RPS + t * _RT
        x_off = pl.multiple_of(row0 * _NFP, _IE)
        pltpu.sync_copy(xp_h.at[pl.ds(x_off, _IE)], xf_v)

        def bld_ie(i, c2):
            s = pl.ds(pl.multiple_of(i * 16, 16), 16)
            xo = xf_v[s] + offs_v[s]
            ie_v[s] = xo
            iq_v[s] = xo >> 4
            return c2

        lax.fori_loop(0, _IE // 16, bld_ie, 0)

        def bld_iab(i, c2):
            s = pl.ds(pl.multiple_of(i * 16, 16), 16)
            iab_v[s] = plsc.load_gather(xf_v, [cols_v[s]]) + adds_v[s]
            return c2

        lax.fori_loop(0, _IAB // 16, bld_iab, 0)

        cps = []
        for c in range(_NCH):
            cps.append(pltpu.async_copy(
                ffm_f.at[iab_v.at[pl.ds(c * _CH, _CH)]],
                buf_v.at[pl.ds(c * _CH, _CH)], sem))
        cps.append(pltpu.async_copy(emb_h.at[ie_v], ebuf_v, sem2))
        cps.append(pltpu.async_copy(lw16_h.at[iq_v], lbuf_v, sem2))
        for cp in cps:
            cp.wait()

        for r in range(_RT):
            base_in = r * 2 * _NPP
            base_out = r * _NP * _ED

            def prod(i, c2, base_in=base_in, base_out=base_out):
                a = buf_v[base_in + 2 * i, :]
                b = buf_v[base_in + 2 * i + 1, :]
                o = pl.ds(pl.multiple_of(base_out + i * _ED, _ED), _ED)
                out_v[o] = a * b
                return c2

            lax.fori_loop(0, _NP, prod, 0)

        int_off = pl.multiple_of(row0 * _NP * _ED, _RT * _NP * _ED)
        pltpu.sync_copy(out_v, inter_o.at[pl.ds(int_off, _RT * _NP * _ED)])
        ie_off = pl.multiple_of(row0 * _NFP, _IE)
        pltpu.sync_copy(ebuf_v, hrows_o.at[pl.ds(ie_off, _IE)])
        pltpu.sync_copy(lbuf_v, lwg_o.at[pl.ds(ie_off, _IE)])
        return carry

    lax.fori_loop(0, _STEPS, step, 0)


_sc_call_cache = []


def _sc_call(*args):
    if not _sc_call_cache:
        _sc_call_cache.append(pl.kernel(
            _sc_body,
            out_type=[
                jax.ShapeDtypeStruct((_B * _NP * _ED,), jnp.float32),
                jax.ShapeDtypeStruct((_B * _NFP, _ED), jnp.float32),
                jax.ShapeDtypeStruct((_B * _NFP, _ED), jnp.float32),
            ],
            mesh=plsc.VectorSubcoreMesh(core_axis_name="c",
                                        subcore_axis_name="s"),
            compiler_params=pltpu.CompilerParams(use_tc_tiling_on_sc=False,
                                                 needs_layout_passes=False),
            scratch_types=[
                pltpu.VMEM((_IAB,), jnp.int32),
                pltpu.VMEM((_IAB,), jnp.int32),
                pltpu.VMEM((_IE,), jnp.int32),
                pltpu.VMEM((_IE,), jnp.int32),
                pltpu.VMEM((_IAB,), jnp.int32),
                pltpu.VMEM((_IE,), jnp.int32),
                pltpu.VMEM((_IE,), jnp.int32),
                pltpu.VMEM((_IAB, _ED), jnp.float32),
                pltpu.VMEM((_RT * _NP * _ED,), jnp.float32),
                pltpu.VMEM((_IE, _ED), jnp.float32),
                pltpu.VMEM((_IE, _ED), jnp.float32),
                pltpu.SemaphoreType.DMA,
                pltpu.SemaphoreType.DMA,
            ],
        ))
    return _sc_call_cache[0](*args)


_BB = 512


def _tc_body(inter_r, h_r, lwr_r, oh_r, lb_r,
             w1_r, b1_r, w2_r, b2_r,
             wf1f_r, wf1i_r, wf1d_r, bf1_r,
             wf2_r, bf2_r, wo_r, bo_r, out_r):
    dot = functools.partial(jnp.dot, preferred_element_type=jnp.float32,
                            precision=lax.Precision.HIGHEST)
    first = (jnp.sum(lwr_r[...] * oh_r[...], axis=1, keepdims=True)
             + lb_r[0, 0])
    d1 = jnp.maximum(dot(h_r[...], w1_r[...]) + b1_r[...], 0.0)
    d2 = jnp.maximum(dot(d1, w2_r[...]) + b2_r[...], 0.0)
    t = (dot(inter_r[...], wf1i_r[...]) + dot(d2, wf1d_r[...])
         + first * wf1f_r[...] + bf1_r[...])
    h2 = jnp.maximum(t, 0.0)
    h3 = jnp.maximum(dot(h2, wf2_r[...]) + bf2_r[...], 0.0)
    out_r[...] = dot(h3, wo_r[...]) + bo_r[...]


def _full(shape):
    return pl.BlockSpec(shape, lambda i: (0, 0))


_tc_call = pl.pallas_call(
    _tc_body,
    grid=(_B // _BB,),
    in_specs=[
        pl.BlockSpec((_BB, _NP * _ED), lambda i: (i, 0)),
        pl.BlockSpec((_BB, _NFP * _ED), lambda i: (i, 0)),
        pl.BlockSpec((_BB, _NFP * _ED), lambda i: (i, 0)),
        pl.BlockSpec((_BB, _NFP * _ED), lambda i: (i, 0)),
        _full((1, 1)),
        _full((_NFP * _ED, 64)), _full((1, 64)),
        _full((64, 64)), _full((1, 64)),
        _full((1, 64)), _full((_NP * _ED, 64)), _full((64, 64)), _full((1, 64)),
        _full((64, 32)), _full((1, 32)),
        _full((32, 1)), _full((1, 1)),
    ],
    out_specs=pl.BlockSpec((_BB, 1), lambda i: (i, 0)),
    out_shape=jax.ShapeDtypeStruct((_B, 1), jnp.float32),
)


def _fold_bn(W, b, g, be, m, v):
    s = g * lax.rsqrt(v + 1e-5)
    return W * s[None, :], ((b - m) * s + be)[None, :]


def kernel(x, linear_w, linear_b, emb, ffm,
           W1, b1, g1, be1, m1, v1, W2, b2, g2, be2, m2, v2,
           Wf1, bf1, gf1, bef1, mf1, vf1, Wf2, bf2, gf2, bef2, mf2, vf2,
           Wout, bout):
    xp = jnp.pad(x, ((0, 0), (0, _NFP - _NF))).reshape(-1)
    xo = x + jnp.asarray(_OFF, dtype=x.dtype)[None, :]
    ohsmall = (jnp.arange(_ED, dtype=x.dtype)[None, None, :]
               == (xo % _ED)[:, :, None]).astype(jnp.float32)
    oh = jnp.pad(ohsmall, ((0, 0), (0, _NFP - _NF), (0, 0)))
    oh = oh.reshape(_B, _NFP * _ED)
    cols, adds, offs = _static_tables()
    ffm_f = (jnp.swapaxes(ffm, 1, 2)
             .reshape(_NF, _ED, _FD // 8, 8)
             .transpose(0, 2, 3, 1)
             .reshape(_NF * _FD, _ED))
    lw16 = linear_w.reshape(_FD // _ED, _ED)

    inter_f, hrows, lwrows = _sc_call(ffm_f, xp, cols, adds, offs,
                                      emb, lw16)

    W1p, b1p = _fold_bn(W1, b1, g1, be1, m1, v1)
    W1a = jnp.pad(W1p.reshape(_NF, _ED, 64),
                  ((0, _NFP - _NF), (0, 0), (0, 0))).reshape(_NFP * _ED, 64)
    W2p, b2p = _fold_bn(W2, b2, g2, be2, m2, v2)
    Wf1p, bf1p = _fold_bn(Wf1, bf1, gf1, bef1, mf1, vf1)
    Wf2p, bf2p = _fold_bn(Wf2, bf2, gf2, bef2, mf2, vf2)

    out2d = _tc_call(
        inter_f.reshape(_B, _NP * _ED),
        hrows.reshape(_B, _NFP * _ED),
        lwrows.reshape(_B, _NFP * _ED),
        oh,
        linear_b.reshape(1, 1),
        W1a, b1p, W2p, b2p,
        Wf1p[0:1, :], Wf1p[1:1 + _NP * _ED, :], Wf1p[1 + _NP * _ED:, :], bf1p,
        Wf2p, bf2p, Wout, bout.reshape(1, 1),
    )
    return out2d[:, 0]


# speedup vs baseline: 2.3265x; 1.6687x over previous
import functools

import numpy as np
import jax
import jax.numpy as jnp
from jax import lax
from jax.experimental import pallas as pl
from jax.experimental.pallas import tpu as pltpu
from jax.experimental.pallas import tpu_sc as plsc

_NF = 26
_NFP = 32
_ED = 16
_FD = 260000
_B = 4096
_NP = (_NF * (_NF - 1)) // 2
_NPP = 336
_OFF = np.arange(_NF, dtype=np.int32) * 10000

_pairs = np.array([(f, g) for f in range(_NF - 1) for g in range(f + 1, _NF)],
                  dtype=np.int32)
_PF = _pairs[:, 0]
_PG = _pairs[:, 1]

_NW = 32
_RPS = _B // _NW
_RT = 4
_STEPS = _RPS // _RT
_IAB = _RT * 2 * _NPP
_CH = 128
_NCH = _IAB // _CH
_IE = _RT * _NFP


def _static_tables():
    col1 = np.zeros(2 * _NPP, np.int32)
    add1 = np.zeros(2 * _NPP, np.int32)
    col1[0:650:2] = _PG
    add1[0:650:2] = _OFF[_PG] + _PF * _FD
    col1[1:650:2] = _PF
    add1[1:650:2] = _OFF[_PF] + _PG * _FD
    cols = np.concatenate([col1 + 32 * r for r in range(_RT)])
    adds = np.concatenate([add1 for _ in range(_RT)])
    offc = np.zeros(_NFP, np.int32)
    offc[:_NF] = _OFF
    offs = np.tile(offc, _RT)
    return jnp.asarray(cols), jnp.asarray(adds), jnp.asarray(offs)


def _sc_body(ffm_f, xp_h, cols_h, adds_h, offs_h, emb_h, lw16_h,
             inter_o, hrows_o, lwg_o,
             cols_v, adds_v, offs_v,
             xf_v, iab_v, ie_v, iq_v, buf_v, out_v, ebuf_v, lbuf_v,
             sem, sem2):
    nc = 2
    wid = lax.axis_index("s") * nc + lax.axis_index("c")
    pltpu.sync_copy(cols_h, cols_v)
    pltpu.sync_copy(adds_h, adds_v)
    pltpu.sync_copy(offs_h, offs_v)

    def step(t, carry):
        row0 = wid * _RPS + t * _RT
        x_off = pl.multiple_of(row0 * _NFP, _IE)
        pltpu.sync_copy(xp_h.at[pl.ds(x_off, _IE)], xf_v)

        def bld_ie(i, c2):
            s = pl.ds(pl.multiple_of(i * 16, 16), 16)
            xo = xf_v[s] + offs_v[s]
            ie_v[s] = xo
            iq_v[s] = xo >> 4
            return c2

        lax.fori_loop(0, _IE // 16, bld_ie, 0)

        def bld_iab(i, c2):
            s = pl.ds(pl.multiple_of(i * 16, 16), 16)
            iab_v[s] = plsc.load_gather(xf_v, [cols_v[s]]) + adds_v[s]
            return c2

        lax.fori_loop(0, _IAB // 16, bld_iab, 0)

        cps = []
        for c in range(_NCH):
            cps.append(pltpu.async_copy(
                ffm_f.at[iab_v.at[pl.ds(c * _CH, _CH)]],
                buf_v.at[pl.ds(c * _CH, _CH)], sem))
        cps.append(pltpu.async_copy(emb_h.at[ie_v], ebuf_v, sem2))
        cps.append(pltpu.async_copy(lw16_h.at[iq_v], lbuf_v, sem2))
        for cp in cps:
            cp.wait()

        for r in range(_RT):
            base_in = r * 2 * _NPP
            base_out = r * _NP * _ED

            def prod(i, c2, base_in=base_in, base_out=base_out):
                a = buf_v[base_in + 2 * i, :]
                b = buf_v[base_in + 2 * i + 1, :]
                o = pl.ds(pl.multiple_of(base_out + i * _ED, _ED), _ED)
                out_v[o] = a * b
                return c2

            lax.fori_loop(0, _NP, prod, 0)

        int_off = pl.multiple_of(row0 * _NP * _ED, _RT * _NP * _ED)
        pltpu.sync_copy(out_v, inter_o.at[pl.ds(int_off, _RT * _NP * _ED)])
        ie_off = pl.multiple_of(row0 * _NFP, _IE)
        pltpu.sync_copy(ebuf_v, hrows_o.at[pl.ds(ie_off, _IE)])
        pltpu.sync_copy(lbuf_v, lwg_o.at[pl.ds(ie_off, _IE)])
        return carry

    lax.fori_loop(0, _STEPS, step, 0)


_sc_call_cache = []


def _sc_call(*args):
    if not _sc_call_cache:
        _sc_call_cache.append(pl.kernel(
            _sc_body,
            out_type=[
                jax.ShapeDtypeStruct((_B * _NP * _ED,), jnp.float32),
                jax.ShapeDtypeStruct((_B * _NFP, _ED), jnp.float32),
                jax.ShapeDtypeStruct((_B * _NFP, _ED), jnp.float32),
            ],
            mesh=plsc.VectorSubcoreMesh(core_axis_name="c",
                                        subcore_axis_name="s"),
            compiler_params=pltpu.CompilerParams(use_tc_tiling_on_sc=False,
                                                 needs_layout_passes=False),
            scratch_types=[
                pltpu.VMEM((_IAB,), jnp.int32),
                pltpu.VMEM((_IAB,), jnp.int32),
                pltpu.VMEM((_IE,), jnp.int32),
                pltpu.VMEM((_IE,), jnp.int32),
                pltpu.VMEM((_IAB,), jnp.int32),
                pltpu.VMEM((_IE,), jnp.int32),
                pltpu.VMEM((_IE,), jnp.int32),
                pltpu.VMEM((_IAB, _ED), jnp.float32),
                pltpu.VMEM((_RT * _NP * _ED,), jnp.float32),
                pltpu.VMEM((_IE, _ED), jnp.float32),
                pltpu.VMEM((_IE, _ED), jnp.float32),
                pltpu.SemaphoreType.DMA,
                pltpu.SemaphoreType.DMA,
            ],
        ))
    return _sc_call_cache[0](*args)


_TW = 1024
_TJ = _FD // _TW
_TTAIL = _FD - _TJ * _TW
_TU = _NF * (_TJ + 1)
_TT32 = 32


def _sct_body(fft2_h, tail_h, out_o, in_v, out_v):
    nc = 2
    wid = lax.axis_index("s") * nc + lax.axis_index("c")
    iot = lax.iota(jnp.int32, 16)

    @pl.when(wid == 0)
    def _copy_tail():
        def tcp(f, c2):
            src = pl.multiple_of(f * (_ED * _TT32), _ED * _TT32)
            dst = pl.multiple_of((f * _FD + _TJ * _TW + 896) * _ED, 512)
            pltpu.sync_copy(tail_h.at[pl.ds(src, _ED * _TT32)],
                            out_o.at[pl.ds(dst, _ED * _TT32)])
            return c2
        lax.fori_loop(0, _NF, tcp, 0)

    nloop = (_TU + _NW - 1) // _NW

    def unit(i, carry):
        k = wid + i * _NW

        @pl.when(k < _TU)
        def _do():
            f = k // (_TJ + 1)
            j = k % (_TJ + 1)
            c0 = j * _TW

            def run(width):
                pltpu.sync_copy(
                    fft2_h.at[pl.ds(pl.multiple_of(f * _ED, _ED), _ED),
                              pl.ds(pl.multiple_of(c0, _TW), width)],
                    in_v.at[:, pl.ds(0, width)])

                def grp(g, c3):
                    base = iot * 16 + g * 256
                    rs = pl.ds(pl.multiple_of(g * 16, 16), 16)
                    for e in range(_ED):
                        v = in_v[e, rs]
                        plsc.store_scatter(out_v, [base + e], v)
                    return c3

                lax.fori_loop(0, width // 16, grp, 0)
                dst = pl.multiple_of((f * _FD + c0) * _ED, 512)
                pltpu.sync_copy(out_v.at[pl.ds(0, width * _ED)],
                                out_o.at[pl.ds(dst, width * _ED)])

            @pl.when(j < _TJ)
            def _full_w():
                run(_TW)

            @pl.when(j == _TJ)
            def _tail_w():
                run(896)

        return carry

    lax.fori_loop(0, nloop, unit, 0)


_sct_call_cache = []


def _sct_call(fft2, tail):
    if not _sct_call_cache:
        _sct_call_cache.append(pl.kernel(
            _sct_body,
            out_type=jax.ShapeDtypeStruct((_NF * _FD * _ED,), jnp.float32),
            mesh=plsc.VectorSubcoreMesh(core_axis_name="c",
                                        subcore_axis_name="s"),
            compiler_params=pltpu.CompilerParams(use_tc_tiling_on_sc=True,
                                                 needs_layout_passes=False),
            scratch_types=[
                pltpu.VMEM((_ED, _TW), jnp.float32),
                pltpu.VMEM((_TW * _ED,), jnp.float32),
            ],
        ))
    return _sct_call_cache[0](fft2, tail)


_BB = 512


def _tc_body(inter_r, h_r, lwr_r, oh_r, lb_r,
             w1_r, b1_r, w2_r, b2_r,
             wf1f_r, wf1i_r, wf1d_r, bf1_r,
             wf2_r, bf2_r, wo_r, bo_r, out_r):
    dot = functools.partial(jnp.dot, preferred_element_type=jnp.float32,
                            precision=lax.Precision.HIGHEST)
    first = (jnp.sum(lwr_r[...] * oh_r[...], axis=1, keepdims=True)
             + lb_r[0, 0])
    d1 = jnp.maximum(dot(h_r[...], w1_r[...]) + b1_r[...], 0.0)
    d2 = jnp.maximum(dot(d1, w2_r[...]) + b2_r[...], 0.0)
    t = (dot(inter_r[...], wf1i_r[...]) + dot(d2, wf1d_r[...])
         + first * wf1f_r[...] + bf1_r[...])
    h2 = jnp.maximum(t, 0.0)
    h3 = jnp.maximum(dot(h2, wf2_r[...]) + bf2_r[...], 0.0)
    out_r[...] = dot(h3, wo_r[...]) + bo_r[...]


def _full(shape):
    return pl.BlockSpec(shape, lambda i: (0, 0))


_tc_call = pl.pallas_call(
    _tc_body,
    grid=(_B // _BB,),
    in_specs=[
        pl.BlockSpec((_BB, _NP * _ED), lambda i: (i, 0)),
        pl.BlockSpec((_BB, _NFP * _ED), lambda i: (i, 0)),
        pl.BlockSpec((_BB, _NFP * _ED), lambda i: (i, 0)),
        pl.BlockSpec((_BB, _NFP * _ED), lambda i: (i, 0)),
        _full((1, 1)),
        _full((_NFP * _ED, 64)), _full((1, 64)),
        _full((64, 64)), _full((1, 64)),
        _full((1, 64)), _full((_NP * _ED, 64)), _full((64, 64)), _full((1, 64)),
        _full((64, 32)), _full((1, 32)),
        _full((32, 1)), _full((1, 1)),
    ],
    out_specs=pl.BlockSpec((_BB, 1), lambda i: (i, 0)),
    out_shape=jax.ShapeDtypeStruct((_B, 1), jnp.float32),
)


def _fold_bn(W, b, g, be, m, v):
    s = g * lax.rsqrt(v + 1e-5)
    return W * s[None, :], ((b - m) * s + be)[None, :]


def kernel(x, linear_w, linear_b, emb, ffm,
           W1, b1, g1, be1, m1, v1, W2, b2, g2, be2, m2, v2,
           Wf1, bf1, gf1, bef1, mf1, vf1, Wf2, bf2, gf2, bef2, mf2, vf2,
           Wout, bout):
    xp = jnp.pad(x, ((0, 0), (0, _NFP - _NF))).reshape(-1)
    xo = x + jnp.asarray(_OFF, dtype=x.dtype)[None, :]
    ohsmall = (jnp.arange(_ED, dtype=x.dtype)[None, None, :]
               == (xo % _ED)[:, :, None]).astype(jnp.float32)
    oh = jnp.pad(ohsmall, ((0, 0), (0, _NFP - _NF), (0, 0)))
    oh = oh.reshape(_B, _NFP * _ED)
    cols, adds, offs = _static_tables()
    fft2 = jnp.swapaxes(ffm, 1, 2).reshape(_NF * _ED, _FD)
    tail1d = ffm[:, _TJ * _TW + 896:, :].reshape(-1)
    ffm_f = _sct_call(fft2, tail1d).reshape(_NF * _FD, _ED)
    lw16 = linear_w.reshape(_FD // _ED, _ED)

    inter_f, hrows, lwrows = _sc_call(ffm_f, xp, cols, adds, offs,
                                      emb, lw16)

    W1p, b1p = _fold_bn(W1, b1, g1, be1, m1, v1)
    W1a = jnp.pad(W1p.reshape(_NF, _ED, 64),
                  ((0, _NFP - _NF), (0, 0), (0, 0))).reshape(_NFP * _ED, 64)
    W2p, b2p = _fold_bn(W2, b2, g2, be2, m2, v2)
    Wf1p, bf1p = _fold_bn(Wf1, bf1, gf1, bef1, mf1, vf1)
    Wf2p, bf2p = _fold_bn(Wf2, bf2, gf2, bef2, mf2, vf2)

    out2d = _tc_call(
        inter_f.reshape(_B, _NP * _ED),
        hrows.reshape(_B, _NFP * _ED),
        lwrows.reshape(_B, _NFP * _ED),
        oh,
        linear_b.reshape(1, 1),
        W1a, b1p, W2p, b2p,
        Wf1p[0:1, :], Wf1p[1:1 + _NP * _ED, :], Wf1p[1 + _NP * _ED:, :], bf1p,
        Wf2p, bf2p, Wout, bout.reshape(1, 1),
    )
    return out2d[:, 0]


# speedup vs baseline: 3.1508x; 1.3543x over previous
import functools

import numpy as np
import jax
import jax.numpy as jnp
from jax import lax
from jax.experimental import pallas as pl
from jax.experimental.pallas import tpu as pltpu
from jax.experimental.pallas import tpu_sc as plsc

_NF = 26
_NFP = 32
_ED = 16
_FD = 260000
_B = 4096
_NP = (_NF * (_NF - 1)) // 2
_NPP = 336
_OFF = np.arange(_NF, dtype=np.int32) * 10000

_pairs = np.array([(f, g) for f in range(_NF - 1) for g in range(f + 1, _NF)],
                  dtype=np.int32)
_PF = _pairs[:, 0]
_PG = _pairs[:, 1]

_NW = 32
_RPS = _B // _NW
_RT = 4
_STEPS = _RPS // _RT
_IAB = _RT * 2 * _NPP
_CH = 128
_NCH = _IAB // _CH
_IE = _RT * _NFP


def _static_tables():
    col1 = np.zeros(2 * _NPP, np.int32)
    add1 = np.zeros(2 * _NPP, np.int32)
    col1[0:650:2] = _PG
    add1[0:650:2] = _OFF[_PG] + _PF * _FD
    col1[1:650:2] = _PF
    add1[1:650:2] = _OFF[_PF] + _PG * _FD
    cols = np.concatenate([col1 + 32 * r for r in range(_RT)])
    adds = np.concatenate([add1 for _ in range(_RT)])
    offc = np.zeros(_NFP, np.int32)
    offc[:_NF] = _OFF
    offs = np.tile(offc, _RT)
    return jnp.asarray(cols), jnp.asarray(adds), jnp.asarray(offs)


def _sc_body(ffm_f, xp_h, cols_h, adds_h, offs_h, emb_h, lw16_h,
             inter_o, hrows_o, lwg_o,
             cols_v, adds_v, offs_v,
             xf_v, iab_v, ie_v, iq_v, buf_v, out_v, ebuf_v, lbuf_v,
             sem, sem2):
    nc = 2
    wid = lax.axis_index("s") * nc + lax.axis_index("c")
    pltpu.sync_copy(cols_h, cols_v)
    pltpu.sync_copy(adds_h, adds_v)
    pltpu.sync_copy(offs_h, offs_v)

    def step(t, carry):
        row0 = wid * _RPS + t * _RT
        x_off = pl.multiple_of(row0 * _NFP, _IE)
        pltpu.sync_copy(xp_h.at[pl.ds(x_off, _IE)], xf_v)

        def bld_ie(i, c2):
            s = pl.ds(pl.multiple_of(i * 16, 16), 16)
            xo = xf_v[s] + offs_v[s]
            ie_v[s] = xo
            iq_v[s] = xo >> 4
            return c2

        lax.fori_loop(0, _IE // 16, bld_ie, 0)

        def bld_iab(i, c2):
            s = pl.ds(pl.multiple_of(i * 16, 16), 16)
            iab_v[s] = plsc.load_gather(xf_v, [cols_v[s]]) + adds_v[s]
            return c2

        lax.fori_loop(0, _IAB // 16, bld_iab, 0)

        cps = []
        for c in range(_NCH):
            cps.append(pltpu.async_copy(
                ffm_f.at[iab_v.at[pl.ds(c * _CH, _CH)]],
                buf_v.at[pl.ds(c * _CH, _CH)], sem))
        cps.append(pltpu.async_copy(emb_h.at[ie_v], ebuf_v, sem2))
        cps.append(pltpu.async_copy(lw16_h.at[iq_v], lbuf_v, sem2))
        for cp in cps:
            cp.wait()

        for r in range(_RT):
            base_in = r * 2 * _NPP
            base_out = r * _NP * _ED

            def prod(i, c2, base_in=base_in, base_out=base_out):
                a = buf_v[base_in + 2 * i, :]
                b = buf_v[base_in + 2 * i + 1, :]
                o = pl.ds(pl.multiple_of(base_out + i * _ED, _ED), _ED)
                out_v[o] = a * b
                return c2

            lax.fori_loop(0, _NP, prod, 0)

        int_off = pl.multiple_of(row0 * _NP * _ED, _RT * _NP * _ED)
        pltpu.sync_copy(out_v, inter_o.at[pl.ds(int_off, _RT * _NP * _ED)])
        ie_off = pl.multiple_of(row0 * _NFP, _IE)
        pltpu.sync_copy(ebuf_v, hrows_o.at[pl.ds(ie_off, _IE)])
        pltpu.sync_copy(lbuf_v, lwg_o.at[pl.ds(ie_off, _IE)])
        return carry

    lax.fori_loop(0, _STEPS, step, 0)


_sc_call_cache = []


def _sc_call(*args):
    if not _sc_call_cache:
        _sc_call_cache.append(pl.kernel(
            _sc_body,
            out_type=[
                jax.ShapeDtypeStruct((_B * _NP * _ED,), jnp.float32),
                jax.ShapeDtypeStruct((_B * _NFP, _ED), jnp.float32),
                jax.ShapeDtypeStruct((_B * _NFP, _ED), jnp.float32),
            ],
            mesh=plsc.VectorSubcoreMesh(core_axis_name="c",
                                        subcore_axis_name="s"),
            compiler_params=pltpu.CompilerParams(use_tc_tiling_on_sc=False,
                                                 needs_layout_passes=False),
            scratch_types=[
                pltpu.VMEM((_IAB,), jnp.int32),
                pltpu.VMEM((_IAB,), jnp.int32),
                pltpu.VMEM((_IE,), jnp.int32),
                pltpu.VMEM((_IE,), jnp.int32),
                pltpu.VMEM((_IAB,), jnp.int32),
                pltpu.VMEM((_IE,), jnp.int32),
                pltpu.VMEM((_IE,), jnp.int32),
                pltpu.VMEM((_IAB, _ED), jnp.float32),
                pltpu.VMEM((_RT * _NP * _ED,), jnp.float32),
                pltpu.VMEM((_IE, _ED), jnp.float32),
                pltpu.VMEM((_IE, _ED), jnp.float32),
                pltpu.SemaphoreType.DMA,
                pltpu.SemaphoreType.DMA,
            ],
        ))
    return _sc_call_cache[0](*args)


_TW = 1536
_TJ = _FD // _TW
_TLAST = 384
_TUA = _NF * _TJ
_TT32 = 32


def _sct_body(fft2_h, tail_h, out_o, in0_v, in1_v, out0_v, out1_v,
              semi, semo):
    nc = 2
    wid = lax.axis_index("s") * nc + lax.axis_index("c")
    iot = lax.iota(jnp.int32, 16)
    inb = (in0_v, in1_v)
    outb = (out0_v, out1_v)

    def src_of(u):
        f = u // _TJ
        c0 = (u % _TJ) * _TW
        return fft2_h.at[pl.ds(pl.multiple_of(f * _ED, _ED), _ED),
                         pl.ds(pl.multiple_of(c0, _TW), _TW)]

    def dst_of(u):
        f = u // _TJ
        c0 = (u % _TJ) * _TW
        return out_o.at[pl.ds(pl.multiple_of((f * _FD + c0) * _ED, 512),
                              _TW * _ED)]

    def compute(iv, ov, width):
        def grp(g, c3):
            base = iot * 16 + g * 256
            rs = pl.ds(pl.multiple_of(g * 16, 16), 16)
            for e in range(_ED):
                v = iv[e, rs]
                plsc.store_scatter(ov, [base + e], v)
            return c3
        lax.fori_loop(0, width // 16, grp, 0)

    @pl.when(wid == 0)
    def _copy_tail():
        def tcp(f, c2):
            src = pl.multiple_of(f * (_ED * _TT32), _ED * _TT32)
            dst = pl.multiple_of((f * _FD + _TJ * _TW + _TLAST) * _ED, 512)
            pltpu.sync_copy(tail_h.at[pl.ds(src, _ED * _TT32)],
                            out_o.at[pl.ds(dst, _ED * _TT32)])
            return c2
        lax.fori_loop(0, _NF, tcp, 0)

    @pl.when(wid < _NF)
    def _phase_b():
        f = wid
        c0 = _TJ * _TW
        pltpu.sync_copy(
            fft2_h.at[pl.ds(pl.multiple_of(f * _ED, _ED), _ED),
                      pl.ds(pl.multiple_of(c0, _TW), _TLAST)],
            in0_v.at[:, pl.ds(0, _TLAST)])
        compute(in0_v, out0_v, _TLAST)
        dst = pl.multiple_of((f * _FD + c0) * _ED, 512)
        pltpu.sync_copy(out0_v.at[pl.ds(0, _TLAST * _ED)],
                        out_o.at[pl.ds(dst, _TLAST * _ED)])

    nloop = (_TUA + _NW - 1) // _NW

    def valid(u):
        return wid + u * _NW < _TUA

    def uidx(u):
        return wid + u * _NW

    @pl.when(valid(0))
    def _pro():
        pltpu.async_copy(src_of(uidx(0)), inb[0], semi)

    def body(i, carry):
        for b in range(2):
            u = i + b

            @pl.when(valid(u + 1))
            def _fire_in(u=u, b=b):
                pltpu.async_copy(src_of(uidx(u + 1)), inb[(b + 1) % 2], semi)

            @pl.when(jnp.logical_and(valid(u), u >= 2))
            def _drain_out(u=u, b=b):
                pltpu.make_async_copy(outb[b], dst_of(uidx(u - 2)),
                                      semo).wait()

            @pl.when(valid(u))
            def _work(u=u, b=b):
                pltpu.make_async_copy(src_of(uidx(u)), inb[b], semi).wait()
                compute(inb[b], outb[b], _TW)
                pltpu.async_copy(outb[b], dst_of(uidx(u)), semo)
        return carry

    lax.fori_loop(0, nloop // 2, lambda i, c: body(2 * i, c), 0)

    pltpu.make_async_copy(outb[0], dst_of(uidx(0)), semo).wait()
    pltpu.make_async_copy(outb[1], dst_of(uidx(0)), semo).wait()


_sct_call_cache = []


def _sct_call(fft2, tail):
    if not _sct_call_cache:
        _sct_call_cache.append(pl.kernel(
            _sct_body,
            out_type=jax.ShapeDtypeStruct((_NF * _FD * _ED,), jnp.float32),
            mesh=plsc.VectorSubcoreMesh(core_axis_name="c",
                                        subcore_axis_name="s"),
            compiler_params=pltpu.CompilerParams(use_tc_tiling_on_sc=True,
                                                 needs_layout_passes=False),
            scratch_types=[
                pltpu.VMEM((_ED, _TW), jnp.float32),
                pltpu.VMEM((_ED, _TW), jnp.float32),
                pltpu.VMEM((_TW * _ED,), jnp.float32),
                pltpu.VMEM((_TW * _ED,), jnp.float32),
                pltpu.SemaphoreType.DMA,
                pltpu.SemaphoreType.DMA,
            ],
        ))
    return _sct_call_cache[0](fft2, tail)


_BB = 512


def _tc_body(inter_r, h_r, lwr_r, oh_r, lb_r,
             w1_r, b1_r, s1_r, t1_r, w2_r, b2_r, s2_r, t2_r,
             wf1i_r, wf1d_r, bf1_r, sf1_r, tf1_r,
             wf2_r, bf2_r, sf2_r, tf2_r, wo_r, bo_r, out_r):
    dot = functools.partial(jnp.dot, preferred_element_type=jnp.float32)
    first = (jnp.sum(lwr_r[...] * oh_r[...], axis=1, keepdims=True)
             + lb_r[0, 0])
    d1 = jnp.maximum((dot(h_r[...], w1_r[...]) + b1_r[...]) * s1_r[...]
                     + t1_r[...], 0.0)
    d2 = jnp.maximum((dot(d1, w2_r[...]) + b2_r[...]) * s2_r[...]
                     + t2_r[...], 0.0)
    comb = jnp.concatenate([first, inter_r[...]], axis=1)
    u = dot(comb, wf1i_r[...]) + dot(d2, wf1d_r[...]) + bf1_r[...]
    h2 = jnp.maximum(u * sf1_r[...] + tf1_r[...], 0.0)
    h3 = jnp.maximum((dot(h2, wf2_r[...]) + bf2_r[...]) * sf2_r[...]
                     + tf2_r[...], 0.0)
    out_r[...] = dot(h3, wo_r[...]) + bo_r[...]


def _full(shape):
    return pl.BlockSpec(shape, lambda i: (0, 0))


_tc_call = pl.pallas_call(
    _tc_body,
    grid=(_B // _BB,),
    in_specs=[
        pl.BlockSpec((_BB, _NP * _ED), lambda i: (i, 0)),
        pl.BlockSpec((_BB, _NFP * _ED), lambda i: (i, 0)),
        pl.BlockSpec((_BB, _NFP * _ED), lambda i: (i, 0)),
        pl.BlockSpec((_BB, _NFP * _ED), lambda i: (i, 0)),
        _full((1, 1)),
        _full((_NFP * _ED, 64)), _full((1, 64)), _full((1, 64)), _full((1, 64)),
        _full((64, 64)), _full((1, 64)), _full((1, 64)), _full((1, 64)),
        _full((1 + _NP * _ED, 64)), _full((64, 64)), _full((1, 64)),
        _full((1, 64)), _full((1, 64)),
        _full((64, 32)), _full((1, 32)), _full((1, 32)), _full((1, 32)),
        _full((32, 1)), _full((1, 1)),
    ],
    out_specs=pl.BlockSpec((_BB, 1), lambda i: (i, 0)),
    out_shape=jax.ShapeDtypeStruct((_B, 1), jnp.float32),
)


def _bn_affine(g, be, m, v):
    s = g / jnp.sqrt(v + 1e-5)
    return s[None, :], (be - m * s)[None, :]


def kernel(x, linear_w, linear_b, emb, ffm,
           W1, b1, g1, be1, m1, v1, W2, b2, g2, be2, m2, v2,
           Wf1, bf1, gf1, bef1, mf1, vf1, Wf2, bf2, gf2, bef2, mf2, vf2,
           Wout, bout):
    xp = jnp.pad(x, ((0, 0), (0, _NFP - _NF))).reshape(-1)
    xo = x + jnp.asarray(_OFF, dtype=x.dtype)[None, :]
    ohsmall = (jnp.arange(_ED, dtype=x.dtype)[None, None, :]
               == (xo % _ED)[:, :, None]).astype(jnp.float32)
    oh = jnp.pad(ohsmall, ((0, 0), (0, _NFP - _NF), (0, 0)))
    oh = oh.reshape(_B, _NFP * _ED)
    cols, adds, offs = _static_tables()
    fft2 = jnp.swapaxes(ffm, 1, 2).reshape(_NF * _ED, _FD)
    tail1d = ffm[:, _TJ * _TW + _TLAST:, :].reshape(-1)
    ffm_f = _sct_call(fft2, tail1d).reshape(_NF * _FD, _ED)
    lw16 = linear_w.reshape(_FD // _ED, _ED)

    inter_f, hrows, lwrows = _sc_call(ffm_f, xp, cols, adds, offs,
                                      emb, lw16)

    s1, t1 = _bn_affine(g1, be1, m1, v1)
    s2, t2 = _bn_affine(g2, be2, m2, v2)
    sf1, tf1 = _bn_affine(gf1, bef1, mf1, vf1)
    sf2, tf2 = _bn_affine(gf2, bef2, mf2, vf2)
    W1a = jnp.pad(W1.reshape(_NF, _ED, 64),
                  ((0, _NFP - _NF), (0, 0), (0, 0))).reshape(_NFP * _ED, 64)

    out2d = _tc_call(
        inter_f.reshape(_B, _NP * _ED),
        hrows.reshape(_B, _NFP * _ED),
        lwrows.reshape(_B, _NFP * _ED),
        oh,
        linear_b.reshape(1, 1),
        W1a, b1.reshape(1, 64), s1, t1,
        W2, b2.reshape(1, 64), s2, t2,
        Wf1[0:1 + _NP * _ED, :], Wf1[1 + _NP * _ED:, :], bf1.reshape(1, 64),
        sf1, tf1,
        Wf2, bf2.reshape(1, 32), sf2, tf2, Wout, bout.reshape(1, 1),
    )
    return out2d[:, 0]
